# baseline (speedup 1.0000x reference)
import functools

import jax
import jax.numpy as jnp
from jax import lax
from jax.experimental import pallas as pl
from jax.experimental.pallas import tpu as pltpu
from jax.experimental.pallas import tpu_sc as plsc

_B = 16384
_A = 4
_NC = 1
_NS = 16
_L = 16
_NW = _NC * _NS
_CHUNK = _B // _NW
_ROWS = _CHUNK * _A // 128


def _sc_body(u_hbm, cho_hbm, pmf_hbm, u_v, cho_v, pmf_v):
    wid = lax.axis_index("s") * _NC + lax.axis_index("c")
    base = wid * _CHUNK
    pltpu.sync_copy(u_hbm.at[pl.ds(base, _CHUNK)], u_v)

    quarter = jnp.full((_L,), 0.25, jnp.float32)
    half = jnp.full((_L,), 0.5, jnp.float32)
    three_q = jnp.full((_L,), 0.75, jnp.float32)
    one = jnp.full((_L,), 1.0, jnp.float32)
    zeros = jnp.zeros((_L,), jnp.int32)
    ones = jnp.full((_L,), 1, jnp.int32)

    @plsc.parallel_loop(0, _ROWS, step=1, unroll=2)
    def fill(r):
        for c in range(128 // _L):
            pmf_v[r, pl.ds(c * _L, _L)] = quarter

    @plsc.parallel_loop(0, _CHUNK, step=_L, unroll=4)
    def body(s):
        uv = u_v[pl.ds(s, _L)]
        c = lax.select(uv > quarter, ones, zeros)
        c = c + lax.select(uv > half, ones, zeros)
        c = c + lax.select(uv > three_q, ones, zeros)
        c = c + lax.select(uv > one, ones, zeros)
        cho_v[pl.ds(s, _L)] = c

    pltpu.sync_copy(cho_v, cho_hbm.at[pl.ds(base, _CHUNK)])
    pltpu.sync_copy(pmf_v, pmf_hbm.at[wid])


_sc_call = functools.partial(
    pl.kernel,
    out_type=(
        jax.ShapeDtypeStruct((_B,), jnp.int32),
        jax.ShapeDtypeStruct((_NW, _ROWS, 128), jnp.float32),
    ),
    mesh=plsc.VectorSubcoreMesh(
        core_axis_name="c", subcore_axis_name="s", num_cores=_NC
    ),
    scratch_types=[
        pltpu.VMEM((_CHUNK,), jnp.float32),
        pltpu.VMEM((_CHUNK,), jnp.int32),
        pltpu.VMEM((_ROWS, 128), jnp.float32),
    ],
)(_sc_body)


def kernel(current_states, u):
    del current_states
    choices, pmf3 = _sc_call(u.reshape(_B))
    pmfs = pmf3.reshape(128, _A, 128).transpose(0, 2, 1).reshape(_B, _A)
    return pmfs, choices


# speedup vs baseline: 1.0317x; 1.0317x over previous
import functools

import jax
import jax.numpy as jnp
from jax import lax
from jax.experimental import pallas as pl
from jax.experimental.pallas import tpu as pltpu
from jax.experimental.pallas import tpu_sc as plsc

_B = 16384
_A = 4
_NC = 1
_NS = 16
_L = 16
_NW = _NC * _NS
_CHUNK = _B // _NW


def _sc_body(u_hbm, cho_hbm, u_v, cho_v):
    wid = lax.axis_index("s") * _NC + lax.axis_index("c")
    base = wid * _CHUNK
    pltpu.sync_copy(u_hbm.at[pl.ds(base, _CHUNK)], u_v)

    quarter = jnp.full((_L,), 0.25, jnp.float32)
    half = jnp.full((_L,), 0.5, jnp.float32)
    three_q = jnp.full((_L,), 0.75, jnp.float32)
    one = jnp.full((_L,), 1.0, jnp.float32)
    zeros = jnp.zeros((_L,), jnp.int32)
    ones = jnp.full((_L,), 1, jnp.int32)

    @plsc.parallel_loop(0, _CHUNK, step=_L, unroll=4)
    def body(s):
        uv = u_v[pl.ds(s, _L)]
        c = lax.select(uv > quarter, ones, zeros)
        c = c + lax.select(uv > half, ones, zeros)
        c = c + lax.select(uv > three_q, ones, zeros)
        c = c + lax.select(uv > one, ones, zeros)
        cho_v[pl.ds(s, _L)] = c

    pltpu.sync_copy(cho_v, cho_hbm.at[pl.ds(base, _CHUNK)])


_sc_choices = functools.partial(
    pl.kernel,
    out_type=jax.ShapeDtypeStruct((_B,), jnp.int32),
    mesh=plsc.VectorSubcoreMesh(
        core_axis_name="c", subcore_axis_name="s", num_cores=_NC
    ),
    scratch_types=[
        pltpu.VMEM((_CHUNK,), jnp.float32),
        pltpu.VMEM((_CHUNK,), jnp.int32),
    ],
    compiler_params=pltpu.CompilerParams(skip_device_barrier=True),
)(_sc_body)


def _tc_pmf_body(pmf_ref):
    pmf_ref[...] = jnp.full((_B * _A // 128, 128), 0.25, jnp.float32)


_tc_pmf = pl.pallas_call(
    _tc_pmf_body,
    out_shape=jax.ShapeDtypeStruct((_B * _A // 128, 128), jnp.float32),
)


def kernel(current_states, u):
    del current_states
    choices = _sc_choices(u.reshape(_B))
    pmf2 = _tc_pmf()
    pmfs = pmf2.reshape(128, _A, 128).transpose(0, 2, 1).reshape(_B, _A)
    return pmfs, choices
